# baseline (speedup 1.0000x reference)
import functools

import jax
import jax.numpy as jnp
from jax import lax
from jax.experimental import pallas as pl
from jax.experimental.pallas import tpu as pltpu
from jax.experimental.pallas import tpu_sc as plsc

N_NODES = 10000
N_EDGES = 320000
D_FEAT = 128

_NC = 2
_NS = 16
_NW = _NC * _NS
_B_PER_W = N_EDGES // _NW
_CHUNK = 32
_N_FULL = _B_PER_W // _CHUNK
_REM = _B_PER_W - _N_FULL * _CHUNK
_STAGE = 624
_NBUF = 8
_AHEAD = _NBUF - 2
_REV = _N_FULL // _NBUF
assert _N_FULL % _NBUF == 0 and _N_FULL * _CHUNK + _REM == _B_PER_W


def _gather_body(idx_hbm, x_hbm, out_hbm, idx_v, *rest):
    rows = rest[:_NBUF]
    x_sp = rest[_NBUF]
    gsem = rest[_NBUF + 1:2 * _NBUF + 1]
    osem = rest[2 * _NBUF + 1:]
    sid = lax.axis_index("s")
    wid = sid * _NC + lax.axis_index("c")
    base = wid * _B_PER_W

    pltpu.sync_copy(x_hbm.at[pl.ds(sid * _STAGE, _STAGE), :],
                    x_sp.at[pl.ds(sid * _STAGE, _STAGE), :])

    @pl.when(sid == _NS - 1)
    def _():
        pltpu.sync_copy(x_hbm.at[pl.ds(_NS * _STAGE, N_NODES - _NS * _STAGE), :],
                        x_sp.at[pl.ds(_NS * _STAGE, N_NODES - _NS * _STAGE), :])

    pltpu.sync_copy(idx_hbm.at[pl.ds(base, _B_PER_W)], idx_v)
    plsc.subcore_barrier()

    def _src(b):
        return x_hbm if b == _NBUF - 1 else x_sp

    def start_gather(j, b):
        pltpu.async_copy(
            _src(b).at[idx_v.at[pl.ds(j * _CHUNK, _CHUNK)]], rows[b], gsem[b]
        )

    def wait_gather(j, b):
        pltpu.make_async_copy(
            _src(b).at[idx_v.at[pl.ds(j * _CHUNK, _CHUNK)]], rows[b], gsem[b]
        ).wait()

    def start_out(j, b):
        pltpu.async_copy(rows[b], out_hbm.at[pl.ds(base + j * _CHUNK, _CHUNK), :],
                         osem[b])

    def wait_out(j, b):
        pltpu.make_async_copy(rows[b], out_hbm.at[pl.ds(base + j * _CHUNK, _CHUNK), :],
                              osem[b]).wait()

    for j in range(_AHEAD):
        start_gather(j, j)
    for j in range(_NBUF):
        wait_gather(j, j)
        start_out(j, j)
        if j < 2:
            start_gather(j + _AHEAD, (j + _AHEAD) % _NBUF)
        else:
            wait_out(j - 2, (j - 2) % _NBUF)
            start_gather(j + _AHEAD, (j + _AHEAD) % _NBUF)

    def body(h, _):
        for i in range(_NBUF):
            j = _NBUF * h + i
            wait_gather(j, i)
            start_out(j, i)
            wait_out(j - 2, (i - 2) % _NBUF)
            start_gather(j + _AHEAD, (i + _AHEAD) % _NBUF)
        return 0

    lax.fori_loop(1, _REV - 1, body, 0)

    jl = (_REV - 1) * _NBUF
    for j in range(jl, _N_FULL):
        i = j % _NBUF
        wait_gather(j, i)
        start_out(j, i)
        if j + _AHEAD < _N_FULL:
            wait_out(j - 2, (i - 2) % _NBUF)
            start_gather(j + _AHEAD, (i + _AHEAD) % _NBUF)

    wait_out(jl, 0)
    row0 = base + _N_FULL * _CHUNK
    pltpu.async_copy(
        x_sp.at[idx_v.at[pl.ds(_N_FULL * _CHUNK, _REM)]],
        rows[0].at[pl.ds(0, _REM)],
        gsem[0],
    ).wait()
    pltpu.sync_copy(rows[0].at[pl.ds(0, _REM)], out_hbm.at[pl.ds(row0, _REM), :])
    for j in range(jl + 1, _N_FULL):
        wait_out(j, j % _NBUF)


_mesh = plsc.VectorSubcoreMesh(core_axis_name="c", subcore_axis_name="s")

_gather = functools.partial(
    pl.kernel,
    mesh=_mesh,
    out_type=jax.ShapeDtypeStruct((N_EDGES, D_FEAT), jnp.float32),
    scratch_types=[
        pltpu.VMEM((_B_PER_W,), jnp.int32),
    ] + [pltpu.VMEM((_CHUNK, D_FEAT), jnp.float32)] * _NBUF
      + [pltpu.VMEM_SHARED((N_NODES, D_FEAT), jnp.float32)]
      + [pltpu.SemaphoreType.DMA] * (2 * _NBUF),
)(_gather_body)


def kernel(x, edge_index):
    idx_flat = jnp.reshape(edge_index, (2 * N_EDGES,))
    if idx_flat.dtype != jnp.int32:
        idx_flat = idx_flat.astype(jnp.int32)
    return _gather(idx_flat, x)


# speedup vs baseline: 1.1491x; 1.1491x over previous
import functools

import jax
import jax.numpy as jnp
from jax import lax
from jax.experimental import pallas as pl
from jax.experimental.pallas import tpu as pltpu
from jax.experimental.pallas import tpu_sc as plsc

N_NODES = 10000
N_EDGES = 320000
D_FEAT = 128

_NC = 2
_NS = 16
_NW = _NC * _NS
_B_PER_W = N_EDGES // _NW
_CHUNK = 32
_N_FULL = _B_PER_W // _CHUNK
_REM = _B_PER_W - _N_FULL * _CHUNK
_STAGE = 624
_NBUF = 8
_AHEAD = _NBUF - 2
_REV = _N_FULL // _NBUF
assert _N_FULL % _NBUF == 0 and _N_FULL * _CHUNK + _REM == _B_PER_W


def _gather_body(idx_hbm, x_hbm, out_hbm, idx_v, *rest):
    rows = rest[:_NBUF]
    x_sp = rest[_NBUF]
    gsem = rest[_NBUF + 1:2 * _NBUF + 1]
    osem = rest[2 * _NBUF + 1:]
    sid = lax.axis_index("s")
    wid = sid * _NC + lax.axis_index("c")
    base = wid * _B_PER_W

    pltpu.sync_copy(x_hbm.at[pl.ds(sid * _STAGE, _STAGE), :],
                    x_sp.at[pl.ds(sid * _STAGE, _STAGE), :])

    @pl.when(sid == _NS - 1)
    def _():
        pltpu.sync_copy(x_hbm.at[pl.ds(_NS * _STAGE, N_NODES - _NS * _STAGE), :],
                        x_sp.at[pl.ds(_NS * _STAGE, N_NODES - _NS * _STAGE), :])

    pltpu.sync_copy(idx_hbm.at[pl.ds(base, _B_PER_W)], idx_v)
    plsc.subcore_barrier()

    def start_gather(j, b):
        pltpu.async_copy(
            x_sp.at[idx_v.at[pl.ds(j * _CHUNK, _CHUNK)]], rows[b], gsem[b]
        )

    def wait_gather(j, b):
        pltpu.make_async_copy(
            x_sp.at[idx_v.at[pl.ds(j * _CHUNK, _CHUNK)]], rows[b], gsem[b]
        ).wait()

    def start_out(j, b):
        pltpu.async_copy(rows[b], out_hbm.at[pl.ds(base + j * _CHUNK, _CHUNK), :],
                         osem[b])

    def wait_out(j, b):
        pltpu.make_async_copy(rows[b], out_hbm.at[pl.ds(base + j * _CHUNK, _CHUNK), :],
                              osem[b]).wait()

    for j in range(_AHEAD):
        start_gather(j, j)
    for j in range(_NBUF):
        wait_gather(j, j)
        start_out(j, j)
        if j < 2:
            start_gather(j + _AHEAD, (j + _AHEAD) % _NBUF)
        else:
            wait_out(j - 2, (j - 2) % _NBUF)
            start_gather(j + _AHEAD, (j + _AHEAD) % _NBUF)

    def body(h, _):
        for i in range(_NBUF):
            j = _NBUF * h + i
            wait_gather(j, i)
            start_out(j, i)
            wait_out(j - 2, (i - 2) % _NBUF)
            start_gather(j + _AHEAD, (i + _AHEAD) % _NBUF)
        return 0

    lax.fori_loop(1, _REV - 1, body, 0)

    jl = (_REV - 1) * _NBUF
    for j in range(jl, _N_FULL):
        i = j % _NBUF
        wait_gather(j, i)
        start_out(j, i)
        if j + _AHEAD < _N_FULL:
            wait_out(j - 2, (i - 2) % _NBUF)
            start_gather(j + _AHEAD, (i + _AHEAD) % _NBUF)

    wait_out(jl, 0)
    row0 = base + _N_FULL * _CHUNK
    pltpu.async_copy(
        x_sp.at[idx_v.at[pl.ds(_N_FULL * _CHUNK, _REM)]],
        rows[0].at[pl.ds(0, _REM)],
        gsem[0],
    ).wait()
    pltpu.sync_copy(rows[0].at[pl.ds(0, _REM)], out_hbm.at[pl.ds(row0, _REM), :])
    for j in range(jl + 1, _N_FULL):
        wait_out(j, j % _NBUF)


_mesh = plsc.VectorSubcoreMesh(core_axis_name="c", subcore_axis_name="s")

_gather = functools.partial(
    pl.kernel,
    mesh=_mesh,
    out_type=jax.ShapeDtypeStruct((N_EDGES, D_FEAT), jnp.float32),
    scratch_types=[
        pltpu.VMEM((_B_PER_W,), jnp.int32),
    ] + [pltpu.VMEM((_CHUNK, D_FEAT), jnp.float32)] * _NBUF
      + [pltpu.VMEM_SHARED((N_NODES, D_FEAT), jnp.float32)]
      + [pltpu.SemaphoreType.DMA] * (2 * _NBUF),
)(_gather_body)


def kernel(x, edge_index):
    idx_flat = jnp.reshape(edge_index, (2 * N_EDGES,))
    if idx_flat.dtype != jnp.int32:
        idx_flat = idx_flat.astype(jnp.int32)
    return _gather(idx_flat, x)


# speedup vs baseline: 1.1492x; 1.0000x over previous
import functools

import jax
import jax.numpy as jnp
from jax import lax
from jax.experimental import pallas as pl
from jax.experimental.pallas import tpu as pltpu
from jax.experimental.pallas import tpu_sc as plsc

N_NODES = 10000
N_EDGES = 320000
D_FEAT = 128

_NC = 2
_NS = 16
_NW = _NC * _NS
_B_PER_W = N_EDGES // _NW
_CHUNK = 32
_N_FULL = _B_PER_W // _CHUNK
_REM = _B_PER_W - _N_FULL * _CHUNK
_STAGE = 624
_NBUF = 8
_AHEAD = 4
_DRAIN = _NBUF - _AHEAD
_REV = _N_FULL // _NBUF
assert _N_FULL % _NBUF == 0 and _N_FULL * _CHUNK + _REM == _B_PER_W


def _gather_body(idx_hbm, x_hbm, out_hbm, idx_v, *rest):
    rows = rest[:_NBUF]
    x_sp = rest[_NBUF]
    gsem = rest[_NBUF + 1:2 * _NBUF + 1]
    osem = rest[2 * _NBUF + 1:]
    sid = lax.axis_index("s")
    wid = sid * _NC + lax.axis_index("c")
    base = wid * _B_PER_W

    pltpu.sync_copy(x_hbm.at[pl.ds(sid * _STAGE, _STAGE), :],
                    x_sp.at[pl.ds(sid * _STAGE, _STAGE), :])

    @pl.when(sid == _NS - 1)
    def _():
        pltpu.sync_copy(x_hbm.at[pl.ds(_NS * _STAGE, N_NODES - _NS * _STAGE), :],
                        x_sp.at[pl.ds(_NS * _STAGE, N_NODES - _NS * _STAGE), :])

    pltpu.sync_copy(idx_hbm.at[pl.ds(base, _B_PER_W)], idx_v)
    plsc.subcore_barrier()

    def start_gather(j, b):
        pltpu.async_copy(
            x_sp.at[idx_v.at[pl.ds(j * _CHUNK, _CHUNK)]], rows[b], gsem[b]
        )

    def wait_gather(j, b):
        pltpu.make_async_copy(
            x_sp.at[idx_v.at[pl.ds(j * _CHUNK, _CHUNK)]], rows[b], gsem[b]
        ).wait()

    def start_out(j, b):
        pltpu.async_copy(rows[b], out_hbm.at[pl.ds(base + j * _CHUNK, _CHUNK), :],
                         osem[b])

    def wait_out(j, b):
        pltpu.make_async_copy(rows[b], out_hbm.at[pl.ds(base + j * _CHUNK, _CHUNK), :],
                              osem[b]).wait()

    for j in range(_AHEAD):
        start_gather(j, j)
    for j in range(_NBUF):
        wait_gather(j, j)
        start_out(j, j)
        if j < _DRAIN:
            start_gather(j + _AHEAD, (j + _AHEAD) % _NBUF)
        else:
            wait_out(j - _DRAIN, (j + _AHEAD) % _NBUF)
            start_gather(j + _AHEAD, (j + _AHEAD) % _NBUF)

    def body(h, _):
        for i in range(_NBUF):
            j = _NBUF * h + i
            wait_gather(j, i)
            start_out(j, i)
            wait_out(j - _DRAIN, (i + _AHEAD) % _NBUF)
            start_gather(j + _AHEAD, (i + _AHEAD) % _NBUF)
        return 0

    lax.fori_loop(1, _REV - 1, body, 0)

    jl = (_REV - 1) * _NBUF
    for j in range(jl, _N_FULL):
        i = j % _NBUF
        wait_gather(j, i)
        start_out(j, i)
        if j + _AHEAD < _N_FULL:
            wait_out(j - _DRAIN, (i + _AHEAD) % _NBUF)
            start_gather(j + _AHEAD, (i + _AHEAD) % _NBUF)

    wait_out(jl, 0)
    row0 = base + _N_FULL * _CHUNK
    pltpu.async_copy(
        x_sp.at[idx_v.at[pl.ds(_N_FULL * _CHUNK, _REM)]],
        rows[0].at[pl.ds(0, _REM)],
        gsem[0],
    ).wait()
    pltpu.sync_copy(rows[0].at[pl.ds(0, _REM)], out_hbm.at[pl.ds(row0, _REM), :])
    for j in range(jl + 1, _N_FULL):
        wait_out(j, j % _NBUF)


_mesh = plsc.VectorSubcoreMesh(core_axis_name="c", subcore_axis_name="s")

_gather = functools.partial(
    pl.kernel,
    mesh=_mesh,
    out_type=jax.ShapeDtypeStruct((N_EDGES, D_FEAT), jnp.float32),
    scratch_types=[
        pltpu.VMEM((_B_PER_W,), jnp.int32),
    ] + [pltpu.VMEM((_CHUNK, D_FEAT), jnp.float32)] * _NBUF
      + [pltpu.VMEM_SHARED((N_NODES, D_FEAT), jnp.float32)]
      + [pltpu.SemaphoreType.DMA] * (2 * _NBUF),
)(_gather_body)


def kernel(x, edge_index):
    idx_flat = jnp.reshape(edge_index, (2 * N_EDGES,))
    if idx_flat.dtype != jnp.int32:
        idx_flat = idx_flat.astype(jnp.int32)
    return _gather(idx_flat, x)
